# baseline (device time: 372461 ns/iter reference)
import numpy as np
import jax
import jax.numpy as jnp
from jax import lax
from jax.experimental import pallas as pl
from jax.experimental.pallas import tpu as pltpu

N_DEV = 32
M_CHUNK = 128
WIRE_DTYPE = jnp.bfloat16

CW, CCW = 0, 1
STREAMS = ((CW, 0), (CCW, 2), (CW, 1), (CCW, 3))
N_STREAMS = len(STREAMS)


def _ring_perm():
    import distributed_mesh_v7x as dm

    mesh = dm.get_mesh("i", world_size=N_DEV)
    coord_to_idx = {}
    for i, d in enumerate(mesh.devices.flat):
        coord_to_idx[tuple(getattr(d, "coords", (i,)))] = i
    cyc = [(0, 0), (0, 1), (0, 2), (0, 3), (1, 3), (1, 2), (1, 1), (2, 1),
           (2, 2), (2, 3), (3, 3), (3, 2), (3, 1), (3, 0), (2, 0), (1, 0)]
    ring_coords = [(0, y, z) for (y, z) in cyc]
    ring_coords += [(1, y, z) for (y, z) in reversed(cyc)]
    try:
        ring = np.array([coord_to_idx[c] for c in ring_coords], dtype=np.int32)
    except KeyError:
        ring = np.arange(N_DEV, dtype=np.int32)
    ring_pos = np.zeros(N_DEV, dtype=np.int32)
    ring_pos[ring] = np.arange(N_DEV, dtype=np.int32)
    return ring, ring_pos


def kernel(x, w_mat):
    ring_np, ring_pos_np = _ring_perm()
    m_total, k = x.shape
    _, n = w_mat.shape
    assert m_total == N_DEV * M_CHUNK
    nq = n // 4

    def body(ring_ref, ring_pos_ref, x_ref, w_ref, out_ref,
             send_buf, recv_buf, amax_buf, send_sems, recv_sems,
             ag_send_sems, ag_recv_sems, credit_sems):
        def ring(p):
            return ring_ref[p % N_DEV]

        my_i = lax.axis_index("i")
        my_r = ring_pos_ref[my_i]
        left = ring(my_r - 1)
        right = ring(my_r + 1)

        barrier_sem = pltpu.get_barrier_semaphore()
        for nbr in (left, right):
            pl.semaphore_signal(barrier_sem, inc=1, device_id=(nbr,),
                                device_id_type=pl.DeviceIdType.MESH)
        pl.semaphore_wait(barrier_sem, 2)

        def chunk_dev(j, s):
            direction, _ = STREAMS[j]
            return ring(my_r - 1 - s) if direction == CW else ring(my_r + 1 + s)

        def partial(d, j):
            _, q = STREAMS[j]
            xs = x_ref[pl.ds(d * M_CHUNK, M_CHUNK), :]
            return jnp.dot(xs, w_ref[:, q * nq:(q + 1) * nq],
                           preferred_element_type=jnp.float32)

        def rdma(j, slot):
            direction, _ = STREAMS[j]
            return pltpu.make_async_remote_copy(
                src_ref=send_buf.at[j, slot], dst_ref=recv_buf.at[j, slot],
                send_sem=send_sems.at[j, slot], recv_sem=recv_sems.at[j, slot],
                device_id=(right if direction == CW else left,),
                device_id_type=pl.DeviceIdType.MESH)

        def producer_nbr(j):
            direction, _ = STREAMS[j]
            return left if direction == CW else right

        def do_hop(s, slot, pslot, do_signal, do_wait):
            p = [partial(chunk_dev(j, s), j) for j in range(N_STREAMS)]
            for j in range(N_STREAMS):
                rdma(j, pslot).wait_recv()
                val = recv_buf[j, pslot, :, :].astype(jnp.float32) + p[j]
                if do_signal:
                    pl.semaphore_signal(credit_sems.at[j], inc=1,
                                        device_id=(producer_nbr(j),),
                                        device_id_type=pl.DeviceIdType.MESH)
                if do_wait:
                    pl.semaphore_wait(credit_sems.at[j], 1)
                    rdma(j, slot).wait_send()
                send_buf[j, slot, :, :] = val.astype(WIRE_DTYPE)
                rdma(j, slot).start()

        for j in range(N_STREAMS):
            send_buf[j, 0, :, :] = partial(chunk_dev(j, 0), j).astype(WIRE_DTYPE)
            rdma(j, 0).start()

        do_hop(1, 1, 0, True, False)
        do_hop(2, 0, 1, True, True)

        def hop_pair(t, carry):
            s = 2 * t + 1
            do_hop(s, 1, 0, True, True)
            do_hop(s + 1, 0, 1, True, True)
            return carry

        lax.fori_loop(1, (N_DEV - 4) // 2, hop_pair, 0)

        do_hop(N_DEV - 3, 1, 0, True, True)
        do_hop(N_DEV - 2, 0, 1, False, True)

        last = (N_DEV - 2) % 2
        p = [partial(my_i, j) for j in range(N_STREAMS)]
        r = []
        for j in range(N_STREAMS):
            rdma(j, last).wait_recv()
            r.append(jnp.maximum(
                recv_buf[j, last, :, :].astype(jnp.float32) + p[j], 0.0))

        for j in range(N_STREAMS):
            rdma(j, 0).wait_send()
            rdma(j, 1).wait_send()

        amax_local = jnp.max(jnp.stack([jnp.max(rj) for rj in r]))
        amax_buf[pl.ds(my_i, 1), :] = jnp.full((1, 128), amax_local,
                                               dtype=jnp.float32)
        def ag_send(koff, carry):
            dst = ring(my_r + koff)
            pltpu.make_async_remote_copy(
                src_ref=amax_buf.at[pl.ds(my_i, 1)],
                dst_ref=amax_buf.at[pl.ds(my_i, 1)],
                send_sem=ag_send_sems.at[koff],
                recv_sem=ag_recv_sems.at[my_i],
                device_id=(dst,), device_id_type=pl.DeviceIdType.MESH,
            ).start()
            return carry

        def ag_wait_recv(koff, carry):
            src = ring(my_r - koff)
            pltpu.make_async_remote_copy(
                src_ref=amax_buf.at[pl.ds(src, 1)],
                dst_ref=amax_buf.at[pl.ds(src, 1)],
                send_sem=ag_send_sems.at[koff],
                recv_sem=ag_recv_sems.at[src],
                device_id=(src,), device_id_type=pl.DeviceIdType.MESH,
            ).wait_recv()
            return carry

        def ag_wait_send(koff, carry):
            pltpu.make_async_remote_copy(
                src_ref=amax_buf.at[pl.ds(my_i, 1)],
                dst_ref=amax_buf.at[pl.ds(my_i, 1)],
                send_sem=ag_send_sems.at[koff],
                recv_sem=ag_recv_sems.at[my_i],
                device_id=(right,), device_id_type=pl.DeviceIdType.MESH,
            ).wait_send()
            return carry

        lax.fori_loop(1, N_DEV, ag_send, 0)
        lax.fori_loop(1, N_DEV, ag_wait_recv, 0)
        lax.fori_loop(1, N_DEV, ag_wait_send, 0)

        amax = jnp.max(amax_buf[:, :])
        scale = jnp.maximum(amax, 1e-30) / 127.0
        for j in range(N_STREAMS):
            _, q = STREAMS[j]
            out_ref[:, q * nq:(q + 1) * nq] = (
                jnp.clip(jnp.round(r[j] / scale), -127.0, 127.0) * scale
            ).astype(jnp.float32)

    return pl.pallas_call(
        body,
        out_shape=jax.ShapeDtypeStruct((M_CHUNK, n), jnp.float32),
        in_specs=[
            pl.BlockSpec(memory_space=pltpu.SMEM),
            pl.BlockSpec(memory_space=pltpu.SMEM),
            pl.BlockSpec(memory_space=pltpu.VMEM),
            pl.BlockSpec(memory_space=pltpu.VMEM),
        ],
        out_specs=pl.BlockSpec(memory_space=pltpu.VMEM),
        scratch_shapes=[
            pltpu.VMEM((N_STREAMS, 2, M_CHUNK, nq), WIRE_DTYPE),
            pltpu.VMEM((N_STREAMS, 2, M_CHUNK, nq), WIRE_DTYPE),
            pltpu.VMEM((N_DEV, 128), jnp.float32),
            pltpu.SemaphoreType.DMA((N_STREAMS, 2)),
            pltpu.SemaphoreType.DMA((N_STREAMS, 2)),
            pltpu.SemaphoreType.DMA((N_DEV,)),
            pltpu.SemaphoreType.DMA((N_DEV,)),
            pltpu.SemaphoreType.REGULAR((N_STREAMS,)),
        ],
        compiler_params=pltpu.CompilerParams(collective_id=0),
    )(jnp.asarray(ring_np), jnp.asarray(ring_pos_np), x, w_mat)
